# baseline (speedup 1.0000x reference)
import functools

import jax
import jax.numpy as jnp
from jax import lax
from jax.experimental import pallas as pl
from jax.experimental.pallas import tpu as pltpu
from jax.experimental.pallas import tpu_sc as plsc

_B = 16384
_F = 32
_ROWS2 = 250000
_NC = 2
_NS = 16
_NW = _NC * _NS
_BPW = _B // _NW
_CHUNK = 128
_NCH = _BPW // _CHUNK
_CPP = 2
_PASS = _CPP * _CHUNK
_L = 16


def _body(user_hbm, movie_hbm, uf_hbm, mf_hbm, out_hbm,
          uidx, midx, urow_ids, mrow_ids, urows, mrows, outv, sem):
    c = lax.axis_index("c")
    s = lax.axis_index("s")
    wid = s * _NC + c
    base = wid * _BPW

    for j in range(_NCH):
        pltpu.sync_copy(user_hbm.at[pl.ds(base + j * _CHUNK, _CHUNK)],
                        uidx.at[j])
        pltpu.sync_copy(movie_hbm.at[pl.ds(base + j * _CHUNK, _CHUNK)],
                        midx.at[j])

        def shift_body(g, _, j=j):
            sl = pl.ds(g * _L, _L)
            urow_ids.at[j][sl] = uidx.at[j][sl] >> 2
            mrow_ids.at[j][sl] = midx.at[j][sl] >> 2
            return 0

        lax.fori_loop(0, _CHUNK // _L, shift_body, 0)

    lane = lax.iota(jnp.int32, _L)

    for p in range(_NCH // _CPP):
        copies = []
        for k in range(_CPP):
            j = p * _CPP + k
            copies.append(pltpu.async_copy(
                uf_hbm.at[urow_ids.at[j]],
                urows.at[pl.ds(k * _CHUNK, _CHUNK)], sem))
            copies.append(pltpu.async_copy(
                mf_hbm.at[mrow_ids.at[j]],
                mrows.at[pl.ds(k * _CHUNK, _CHUNK)], sem))
        for cp in copies:
            cp.wait()

        for k in range(_CPP):
            j = p * _CPP + k

            def group(g, _, j=j, k=k):
                rows16 = (k * _CHUNK + g * _L) + lane
                sl = pl.ds(g * _L, _L)
                ubase = (uidx.at[j][sl] & 3) * _F
                mbase = (midx.at[j][sl] & 3) * _F
                acc = None
                for f in range(_F):
                    uv = plsc.load_gather(urows, [rows16, ubase + f])
                    mv = plsc.load_gather(mrows, [rows16, mbase + f])
                    pr = uv * mv
                    acc = pr if acc is None else acc + pr
                outv[pl.ds(j * _CHUNK + g * _L, _L)] = acc
                return 0

            lax.fori_loop(0, _CHUNK // _L, group, 0)

    pltpu.sync_copy(outv, out_hbm.at[pl.ds(base, _BPW)])


_mesh = plsc.VectorSubcoreMesh(core_axis_name="c", subcore_axis_name="s")

_mf_call = functools.partial(
    pl.kernel,
    out_type=jax.ShapeDtypeStruct((_B,), jnp.float32),
    mesh=_mesh,
    scratch_types=[
        pltpu.VMEM((_NCH, _CHUNK), jnp.int32),
        pltpu.VMEM((_NCH, _CHUNK), jnp.int32),
        pltpu.VMEM((_NCH, _CHUNK), jnp.int32),
        pltpu.VMEM((_NCH, _CHUNK), jnp.int32),
        pltpu.VMEM((_PASS, _CHUNK), jnp.float32),
        pltpu.VMEM((_PASS, _CHUNK), jnp.float32),
        pltpu.VMEM((_BPW,), jnp.float32),
        pltpu.SemaphoreType.DMA,
    ],
    compiler_params=pltpu.CompilerParams(
        needs_layout_passes=False, use_tc_tiling_on_sc=True),
)(_body)


@jax.jit
def kernel(user, movie, user_factors, movie_factors):
    uf2 = user_factors.reshape(_ROWS2, _CHUNK)
    mf2 = movie_factors.reshape(_ROWS2, _CHUNK)
    return _mf_call(user, movie, uf2, mf2)


# speedup vs baseline: 3.6279x; 3.6279x over previous
import functools

import jax
import jax.numpy as jnp
from jax import lax
from jax.experimental import pallas as pl
from jax.experimental.pallas import tpu as pltpu
from jax.experimental.pallas import tpu_sc as plsc

_B = 16384
_F = 32
_N = 1000000
_NC = 2
_NS = 16
_NW = _NC * _NS
_BPW = _B // _NW
_WIN = 128
_BAT = 8
_NBAT = _BPW // _BAT
_L = 16


def _body(user_hbm, movie_hbm, uft_hbm, mft_hbm, out_hbm,
          uidx, midx, uwin, mwin, outv, sem):
    c = lax.axis_index("c")
    s = lax.axis_index("s")
    wid = s * _NC + c
    base = wid * _BPW

    pltpu.sync_copy(user_hbm.at[pl.ds(base, _BPW)], uidx.at[pl.ds(0, _BPW)])
    pltpu.sync_copy(movie_hbm.at[pl.ds(base, _BPW)], midx.at[pl.ds(0, _BPW)])
    zeros16 = jnp.zeros((_L,), jnp.int32)
    uidx[pl.ds(_BPW, _L)] = zeros16
    midx[pl.ds(_BPW, _L)] = zeros16

    lane = lax.iota(jnp.int32, _L)

    def batch(g, rv):
        u16 = uidx[pl.ds(g * _BAT, _L)]
        m16 = midx[pl.ds(g * _BAT, _L)]

        for l in range(_BAT):
            u = u16[l]
            m = m16[l]
            uoff = pl.multiple_of((u >> 7) * _WIN, _WIN)
            moff = pl.multiple_of((m >> 7) * _WIN, _WIN)
            pltpu.make_async_copy(
                uft_hbm.at[:, pl.ds(uoff, _WIN)],
                uwin.at[pl.ds(l * _F, _F)], sem).start()
            pltpu.make_async_copy(
                mft_hbm.at[:, pl.ds(moff, _WIN)],
                mwin.at[pl.ds(l * _F, _F)], sem).start()
        for l in range(2 * _BAT):
            pltpu.make_async_copy(
                uft_hbm.at[:, pl.ds(0, _WIN)],
                uwin.at[pl.ds(0, _F)], sem).wait()

        lbase = (g % 2) * _BAT
        for l in range(_BAT):
            lu = jnp.full((_L,), u16[l] & (_WIN - 1), jnp.int32)
            lm = jnp.full((_L,), m16[l] & (_WIN - 1), jnp.int32)
            r0 = l * _F + lane
            r1 = r0 + _L
            u0 = plsc.load_gather(uwin, [r0, lu])
            u1 = plsc.load_gather(uwin, [r1, lu])
            m0 = plsc.load_gather(mwin, [r0, lm])
            m1 = plsc.load_gather(mwin, [r1, lm])
            p = u0 * m0 + u1 * m1
            sval = jnp.sum(p)
            rv = jnp.where(lane == lbase + l, jnp.full((_L,), sval), rv)

        @pl.when(g % 2 == 1)
        def _():
            outv[pl.ds((g - 1) * _BAT, _L)] = rv

        return rv

    lax.fori_loop(0, _NBAT, batch, jnp.zeros((_L,), jnp.float32))

    pltpu.sync_copy(outv, out_hbm.at[pl.ds(base, _BPW)])


_mesh = plsc.VectorSubcoreMesh(core_axis_name="c", subcore_axis_name="s")

_mf_call = functools.partial(
    pl.kernel,
    out_type=jax.ShapeDtypeStruct((_B,), jnp.float32),
    mesh=_mesh,
    scratch_types=[
        pltpu.VMEM((_BPW + _L,), jnp.int32),
        pltpu.VMEM((_BPW + _L,), jnp.int32),
        pltpu.VMEM((_BAT * _F, _WIN), jnp.float32),
        pltpu.VMEM((_BAT * _F, _WIN), jnp.float32),
        pltpu.VMEM((_BPW,), jnp.float32),
        pltpu.SemaphoreType.DMA,
    ],
    compiler_params=pltpu.CompilerParams(
        needs_layout_passes=False, use_tc_tiling_on_sc=True),
)(_body)


@jax.jit
def kernel(user, movie, user_factors, movie_factors):
    return _mf_call(user, movie, user_factors.T, movie_factors.T)


# speedup vs baseline: 3.7241x; 1.0265x over previous
import functools

import jax
import jax.numpy as jnp
from jax import lax
from jax.experimental import pallas as pl
from jax.experimental.pallas import tpu as pltpu
from jax.experimental.pallas import tpu_sc as plsc

_B = 16384
_F = 32
_HF = 16
_NC = 2
_NS = 16
_NW = _NC * _NS
_BPW = _B // _NW
_WIN = 128
_BAT = 8
_NBAT = _BPW // _BAT
_L = 16


def _fire_half(uft_hbm, mft_hbm, ubuf, mbuf, sem, u16, m16, h):
    for l in range(_BAT):
        uoff = pl.multiple_of((u16[l] >> 7) * _WIN, _WIN)
        moff = pl.multiple_of((m16[l] >> 7) * _WIN, _WIN)
        pltpu.make_async_copy(
            uft_hbm.at[pl.ds(h * _HF, _HF), pl.ds(uoff, _WIN)],
            ubuf.at[pl.ds(l * _HF, _HF)], sem).start()
        pltpu.make_async_copy(
            mft_hbm.at[pl.ds(h * _HF, _HF), pl.ds(moff, _WIN)],
            mbuf.at[pl.ds(l * _HF, _HF)], sem).start()


def _drain_half(uft_hbm, ubuf, sem):
    for _ in range(2 * _BAT):
        pltpu.make_async_copy(
            uft_hbm.at[pl.ds(0, _HF), pl.ds(0, _WIN)],
            ubuf.at[pl.ds(0, _HF)], sem).wait()


def _partials(ubuf, mbuf, u16, m16, lane):
    ps = []
    for l in range(_BAT):
        lu = jnp.full((_L,), u16[l] & (_WIN - 1), jnp.int32)
        lm = jnp.full((_L,), m16[l] & (_WIN - 1), jnp.int32)
        r = l * _HF + lane
        uv = plsc.load_gather(ubuf, [r, lu])
        mv = plsc.load_gather(mbuf, [r, lm])
        ps.append(uv * mv)
    return ps


def _body(user_hbm, movie_hbm, uft_hbm, mft_hbm, out_hbm,
          uidx, midx, ua, ma, ub, mb, outv, sema, semb):
    c = lax.axis_index("c")
    s = lax.axis_index("s")
    wid = s * _NC + c
    base = wid * _BPW

    pltpu.sync_copy(user_hbm.at[pl.ds(base, _BPW)], uidx.at[pl.ds(0, _BPW)])
    pltpu.sync_copy(movie_hbm.at[pl.ds(base, _BPW)], midx.at[pl.ds(0, _BPW)])

    lane = lax.iota(jnp.int32, _L)

    def idx16(g):
        return uidx[pl.ds(g * _BAT, _L)], midx[pl.ds(g * _BAT, _L)]

    u0, m0 = idx16(0)
    _fire_half(uft_hbm, mft_hbm, ua, ma, sema, u0, m0, 0)
    _fire_half(uft_hbm, mft_hbm, ub, mb, semb, u0, m0, 1)

    def batch(g, rv):
        u16, m16 = idx16(g)
        un, mn = idx16(g + 1)

        _drain_half(uft_hbm, ua, sema)
        p0 = _partials(ua, ma, u16, m16, lane)
        _fire_half(uft_hbm, mft_hbm, ua, ma, sema, un, mn, 0)

        _drain_half(uft_hbm, ub, semb)
        p1 = _partials(ub, mb, u16, m16, lane)
        _fire_half(uft_hbm, mft_hbm, ub, mb, semb, un, mn, 1)

        lbase = (g % 2) * _BAT
        for l in range(_BAT):
            sval = jnp.sum(p0[l] + p1[l])
            rv = jnp.where(lane == lbase + l, jnp.full((_L,), sval), rv)

        @pl.when(g % 2 == 1)
        def _():
            outv[pl.ds((g - 1) * _BAT, _L)] = rv

        return rv

    rv = lax.fori_loop(0, _NBAT - 1, batch, jnp.zeros((_L,), jnp.float32))

    g = _NBAT - 1
    u16, m16 = idx16(g)
    _drain_half(uft_hbm, ua, sema)
    p0 = _partials(ua, ma, u16, m16, lane)
    _drain_half(uft_hbm, ub, semb)
    p1 = _partials(ub, mb, u16, m16, lane)
    for l in range(_BAT):
        sval = jnp.sum(p0[l] + p1[l])
        rv = jnp.where(lane == _BAT + l, jnp.full((_L,), sval), rv)
    outv[pl.ds((g - 1) * _BAT, _L)] = rv

    pltpu.sync_copy(outv, out_hbm.at[pl.ds(base, _BPW)])


_mesh = plsc.VectorSubcoreMesh(core_axis_name="c", subcore_axis_name="s")

_mf_call = functools.partial(
    pl.kernel,
    out_type=jax.ShapeDtypeStruct((_B,), jnp.float32),
    mesh=_mesh,
    scratch_types=[
        pltpu.VMEM((_BPW + _L,), jnp.int32),
        pltpu.VMEM((_BPW + _L,), jnp.int32),
        pltpu.VMEM((_BAT * _HF, _WIN), jnp.float32),
        pltpu.VMEM((_BAT * _HF, _WIN), jnp.float32),
        pltpu.VMEM((_BAT * _HF, _WIN), jnp.float32),
        pltpu.VMEM((_BAT * _HF, _WIN), jnp.float32),
        pltpu.VMEM((_BPW,), jnp.float32),
        pltpu.SemaphoreType.DMA,
        pltpu.SemaphoreType.DMA,
    ],
    compiler_params=pltpu.CompilerParams(
        needs_layout_passes=False, use_tc_tiling_on_sc=True),
)(_body)


@jax.jit
def kernel(user, movie, user_factors, movie_factors):
    return _mf_call(user, movie, user_factors.T, movie_factors.T)
